# Initial kernel scaffold; baseline (speedup 1.0000x reference)
import jax
import jax.numpy as jnp
from jax.experimental import pallas as pl


def kernel(x, labels, class_means, global_mean):
    raise NotImplementedError("write your pallas kernel here")


# speedup vs baseline: 1.2348x; 1.2348x over previous
import functools

import jax
import jax.numpy as jnp
from jax import lax
from jax.experimental import pallas as pl
from jax.experimental.pallas import tpu as pltpu
from jax.experimental.pallas import tpu_sc as plsc

_B = 16384
_F = 2
_FLAT = _B * _F
_NC = 2
_NS = 16
_NW = _NC * _NS
_CHUNK_S = _B // _NW
_CHUNK_F = _CHUNK_S * _F
_L = 16
_NVEC = _CHUNK_F // _L


def _sc_body(x_hbm, lab_hbm, cm_hbm, out_hbm, x_v, lab_v, cm_v, out_v):
    wid = lax.axis_index("s") * _NC + lax.axis_index("c")
    sbase = wid * _CHUNK_S
    fbase = wid * _CHUNK_F
    pltpu.sync_copy(lab_hbm.at[pl.ds(sbase, _CHUNK_S)], lab_v)
    pltpu.sync_copy(x_hbm.at[pl.ds(fbase, _CHUNK_F)], x_v)
    pltpu.sync_copy(cm_hbm, cm_v)
    iota = lax.iota(jnp.int32, _L)
    half = iota >> 1
    parity = iota & 1
    for j in range(_NVEC):
        lab16 = plsc.load_gather(lab_v, [half + (j * (_L // _F))])
        m16 = plsc.load_gather(cm_v, [lab16 * _F + parity])
        out_v[pl.ds(j * _L, _L)] = x_v[pl.ds(j * _L, _L)] - m16
    pltpu.sync_copy(out_v, out_hbm.at[pl.ds(fbase, _CHUNK_F)])


_sc_call = functools.partial(
    pl.kernel,
    out_type=jax.ShapeDtypeStruct((_FLAT,), jnp.float32),
    mesh=plsc.VectorSubcoreMesh(core_axis_name="c", subcore_axis_name="s"),
    compiler_params=pltpu.CompilerParams(needs_layout_passes=False),
    scratch_types=[
        pltpu.VMEM((_CHUNK_F,), jnp.float32),
        pltpu.VMEM((_CHUNK_S,), jnp.int32),
        pltpu.VMEM((_F * 3,), jnp.float32),
        pltpu.VMEM((_CHUNK_F,), jnp.float32),
    ],
)(_sc_body)


@jax.jit
def kernel(x, labels, class_means, global_mean):
    out_flat = _sc_call(x.reshape(_FLAT), labels, class_means.reshape(_F * 3))
    return out_flat.reshape(_B, _F)
